# baseline (speedup 1.0000x reference)
import jax
import jax.numpy as jnp
from jax import lax
from jax.experimental import pallas as pl
from jax.experimental.pallas import tpu as pltpu

NUM_TOKENS = 32768
D_MODEL = 4096
NUM_EXPERTS = 64

TM = 512
NBUF = 4


def _outer(x_hbm, wt_ref, oh_hbm, top_hbm, logits_hbm):
    wt = wt_ref[...]

    def inner(x_ref, oh_ref, top_ref, logits_ref):
        logits = jnp.dot(x_ref[...], wt, preferred_element_type=jnp.float32)
        m = jnp.max(logits, axis=1, keepdims=True)
        s = jnp.sum(jnp.exp(logits - m), axis=1, keepdims=True)
        ii = lax.broadcasted_iota(jnp.int32, logits.shape, 1)
        cand = jnp.where(logits == m, ii, NUM_EXPERTS)
        first = jnp.min(cand, axis=1, keepdims=True)
        oh_ref[...] = (ii == first).astype(jnp.int32)
        top_ref[...] = (1.0 / s)[:, 0]
        logits_ref[...] = logits

    pipeline = pltpu.emit_pipeline(
        inner,
        grid=(NUM_TOKENS // TM,),
        in_specs=[
            pl.BlockSpec(
                (TM, D_MODEL),
                lambda i: (i, 0),
                pipeline_mode=pl.Buffered(buffer_count=NBUF, use_lookahead=True),
            ),
        ],
        out_specs=[
            pl.BlockSpec((TM, NUM_EXPERTS), lambda i: (i, 0)),
            pl.BlockSpec((TM,), lambda i: (i,)),
            pl.BlockSpec((TM, NUM_EXPERTS), lambda i: (i, 0)),
        ],
        dimension_semantics=(pltpu.PARALLEL,),
    )
    pipeline(x_hbm, oh_hbm, top_hbm, logits_hbm)


@jax.jit
def kernel(x, W):
    wt = W.T
    oh, top, logits = pl.pallas_call(
        _outer,
        in_specs=[
            pl.BlockSpec(memory_space=pl.ANY),
            pl.BlockSpec((D_MODEL, NUM_EXPERTS), lambda: (0, 0)),
        ],
        out_specs=[
            pl.BlockSpec(memory_space=pl.ANY),
            pl.BlockSpec(memory_space=pl.ANY),
            pl.BlockSpec(memory_space=pl.ANY),
        ],
        out_shape=[
            jax.ShapeDtypeStruct((NUM_TOKENS, NUM_EXPERTS), jnp.int32),
            jax.ShapeDtypeStruct((NUM_TOKENS,), jnp.float32),
            jax.ShapeDtypeStruct((NUM_TOKENS, NUM_EXPERTS), jnp.float32),
        ],
    )(x, wt)
    return oh, top.reshape(NUM_TOKENS, 1), logits
